# baseline (speedup 1.0000x reference)
import functools

import jax
import jax.numpy as jnp
from jax import lax
from jax.experimental import pallas as pl
from jax.experimental.pallas import tpu as pltpu
from jax.experimental.pallas import tpu_sc as plsc

_L = 16
_NW = 32
_NPH = 4


def _router_body(num_tokens, d_model, n_tiles,
                 x4, posflat, sl_hbm, sigs_hbm, pe_hbm, plate_hbm,
                 sel_out, tgt_out,
                 xga, xgb, pos_v, sel_v, tgt_v,
                 sigs_v, pe_v, plate_v, sl_v,
                 sem_a, sem_b, sem_p):
  per = num_tokens // _NW
  ngrp = per // 8
  gper = ngrp // _NPH
  tpp = gper * 8
  cpp = tpp // _L

  wid = lax.axis_index("s") * 2 + lax.axis_index("c")
  base = wid * per
  gbase = wid * ngrp

  bufs = (xga, xgb)
  sems = (sem_a, sem_b)

  def fire(ph):
    return pltpu.async_copy(
        x4.at[pl.ds(gbase + ph * gper, gper), :, pl.ds(0, 128)],
        bufs[ph % 2], sems[ph % 2])

  h = [fire(0), fire(1)]
  cp = pltpu.async_copy(posflat.at[pl.ds(base, per)], pos_v, sem_p)
  pltpu.sync_copy(sigs_hbm, sigs_v)
  pltpu.sync_copy(pe_hbm, pe_v)
  pltpu.sync_copy(plate_hbm, plate_v)
  pltpu.sync_copy(sl_hbm, sl_v)

  lane = lax.iota(jnp.int32, _L)
  one_i = jnp.full((_L,), 1, jnp.int32)
  zero_i = jnp.full((_L,), 0, jnp.int32)
  two_i = jnp.full((_L,), 2, jnp.int32)
  four_i = jnp.full((_L,), 4, jnp.int32)
  seven_i = jnp.full((_L,), 7, jnp.int32)
  three_i = jnp.full((_L,), 3, jnp.int32)
  one_f = jnp.full((_L,), 1.0, jnp.float32)
  two_f = jnp.full((_L,), 2.0, jnp.float32)
  zero_f = jnp.full((_L,), 0.0, jnp.float32)
  neg_inf = jnp.full((_L,), -jnp.inf, jnp.float32)

  half = lax.shift_right_logical(sl_v[...] + one_i, one_i)

  gd = lax.GatherDimensionNumbers(
      offset_dims=(), collapsed_slice_dims=(0,), start_index_map=(0,))

  def butterfly(vec, op):
    cur = vec
    for sh in (8, 4, 2, 1):
      perm = lane ^ jnp.full((_L,), sh, jnp.int32)
      partner = lax.gather(cur, perm[:, None], gd, slice_sizes=(1,),
                           mode=lax.GatherScatterMode.PROMISE_IN_BOUNDS)
      cur = op(cur, partner)
    return cur

  svec_e = neg_inf
  svec_l = neg_inf
  w_e = pe_v[...]
  w_l = plate_v[...]
  for t in range(n_tiles):
    row = sigs_v[pl.ds(t * _L, _L)]
    th = one_f - two_f / (jnp.exp(row * two_f) + one_f)
    t_spl = jnp.full((_L,), t, jnp.int32)
    svec_e = jnp.where(lane == t_spl, butterfly(w_e * th, jnp.add), svec_e)
    svec_l = jnp.where(lane == t_spl, butterfly(w_l * th, jnp.add), svec_l)

  def argmax_splat(svec):
    cur = butterfly(svec, jnp.maximum)
    sel = plsc.all_reduce_ffs(svec == cur)
    return jnp.broadcast_to(sel, (_L,))

  e_sel = argmax_splat(svec_e)
  l_sel = argmax_splat(svec_l)

  cp.wait()

  for ph in range(_NPH):
    h[ph % 2].wait()
    buf = bufs[ph % 2]
    for c in range(cpp):
      sl_ix = pl.ds(ph * tpp + c * _L, _L)
      tl = jnp.full((_L,), c * _L, jnp.int32) + lane
      gi = lax.shift_right_logical(tl, three_i)
      ri = tl & seven_i
      x0 = plsc.load_gather(buf, (gi, ri, zero_i))
      x1 = plsc.load_gather(buf, (gi, ri, one_i))
      is_early = pos_v[sl_ix] < half
      sel_v[sl_ix] = jnp.where(is_early, e_sel, l_sel)
      tgt_v[sl_ix] = (jnp.where(is_early, zero_i, four_i)
                      + jnp.where(x0 > zero_f, two_i, zero_i)
                      + jnp.where(x1 > zero_f, one_i, zero_i))
    if ph + 2 < _NPH:
      h[ph % 2] = fire(ph + 2)

  pltpu.sync_copy(sel_v, sel_out.at[pl.ds(base, per)])
  pltpu.sync_copy(tgt_v, tgt_out.at[pl.ds(base, per)])


def kernel(x, positions, seq_len, position_sigs, pos_early, pos_late):
  b, s, d = x.shape
  n = b * s
  t_tiles = position_sigs.shape[0]
  per = n // _NW
  gper = per // 8 // _NPH

  x4 = x.reshape(n // 8, 8, d)
  posflat = positions.reshape(n).astype(jnp.int32)
  sl = jnp.full((_L,), seq_len, dtype=jnp.int32)

  mesh = plsc.VectorSubcoreMesh(core_axis_name="c", subcore_axis_name="s",
                                num_cores=2, num_subcores=16)
  out_i32 = jax.ShapeDtypeStruct((n,), jnp.int32)
  fn = pl.kernel(
      functools.partial(_router_body, n, d, t_tiles),
      out_type=[out_i32, out_i32],
      mesh=mesh,
      compiler_params=pltpu.CompilerParams(needs_layout_passes=False),
      scratch_types=[
          pltpu.VMEM((gper, 8, 128), jnp.float32),
          pltpu.VMEM((gper, 8, 128), jnp.float32),
          pltpu.VMEM((per,), jnp.int32),
          pltpu.VMEM((per,), jnp.int32),
          pltpu.VMEM((per,), jnp.int32),
          pltpu.VMEM((t_tiles * _L,), jnp.float32),
          pltpu.VMEM((_L,), jnp.float32),
          pltpu.VMEM((_L,), jnp.float32),
          pltpu.VMEM((_L,), jnp.int32),
          pltpu.SemaphoreType.DMA,
          pltpu.SemaphoreType.DMA,
          pltpu.SemaphoreType.DMA,
      ],
  )
  sel, tgt = fn(x4, posflat, sl, position_sigs.reshape(t_tiles * 16),
                pos_early, pos_late)
  return sel.reshape(b, s), tgt.reshape(b, s)


# speedup vs baseline: 1.1659x; 1.1659x over previous
import functools

import jax
import jax.numpy as jnp
from jax import lax
from jax.experimental import pallas as pl
from jax.experimental.pallas import tpu as pltpu
from jax.experimental.pallas import tpu_sc as plsc

_L = 16
_NW = 32


def _router_body(batch, seq, d_model, n_tiles,
                 x4, pos2d, half_hbm, tab_hbm,
                 sel_out, tgt_out,
                 xga, xgb, pos_v, sel_v, tgt_v, tab_v, half_v,
                 sem_a, sem_b, sem_p):
  stripe = seq // _NW
  gper = stripe // 8
  cpb = stripe // _L

  wid = lax.axis_index("s") * 2 + lax.axis_index("c")
  col0 = wid * stripe

  bufs = (xga, xgb)
  sems = (sem_a, sem_b)

  def fire(ph):
    return pltpu.async_copy(
        x4.at[pl.ds(ph * (seq // 8) + wid * gper, gper), :, pl.ds(0, 128)],
        bufs[ph % 2], sems[ph % 2])

  h = [fire(0), fire(1)]
  cp = pltpu.async_copy(pos2d.at[:, pl.ds(col0, stripe)], pos_v, sem_p)
  pltpu.sync_copy(tab_hbm, tab_v)
  pltpu.sync_copy(half_hbm, half_v)

  lane = lax.iota(jnp.int32, _L)
  one_i = jnp.full((_L,), 1, jnp.int32)
  zero_i = jnp.full((_L,), 0, jnp.int32)
  two_i = jnp.full((_L,), 2, jnp.int32)
  four_i = jnp.full((_L,), 4, jnp.int32)
  seven_i = jnp.full((_L,), 7, jnp.int32)
  three_i = jnp.full((_L,), 3, jnp.int32)
  one_f = jnp.full((_L,), 1.0, jnp.float32)
  two_f = jnp.full((_L,), 2.0, jnp.float32)
  zero_f = jnp.full((_L,), 0.0, jnp.float32)
  neg_inf = jnp.full((_L,), -jnp.inf, jnp.float32)

  half = half_v[...]

  gd = lax.GatherDimensionNumbers(
      offset_dims=(), collapsed_slice_dims=(0,), start_index_map=(0,))

  def butterfly(vec, op):
    cur = vec
    for sh in (8, 4, 2, 1):
      perm = lane ^ jnp.full((_L,), sh, jnp.int32)
      partner = lax.gather(cur, perm[:, None], gd, slice_sizes=(1,),
                           mode=lax.GatherScatterMode.PROMISE_IN_BOUNDS)
      cur = op(cur, partner)
    return cur

  svec_e = neg_inf
  svec_l = neg_inf
  w_e = tab_v[pl.ds(n_tiles * _L, _L)]
  w_l = tab_v[pl.ds((n_tiles + 1) * _L, _L)]
  for t in range(n_tiles):
    row = tab_v[pl.ds(t * _L, _L)]
    th = one_f - two_f / (jnp.exp(row * two_f) + one_f)
    t_spl = jnp.full((_L,), t, jnp.int32)
    svec_e = jnp.where(lane == t_spl, butterfly(w_e * th, jnp.add), svec_e)
    svec_l = jnp.where(lane == t_spl, butterfly(w_l * th, jnp.add), svec_l)

  def argmax_splat(svec):
    cur = butterfly(svec, jnp.maximum)
    sel = plsc.all_reduce_ffs(svec == cur)
    return jnp.broadcast_to(sel, (_L,))

  e_sel = argmax_splat(svec_e)
  l_sel = argmax_splat(svec_l)

  cp.wait()

  for ph in range(batch):
    h[ph % 2].wait()
    buf = bufs[ph % 2]
    for c in range(cpb):
      sl_ix = pl.ds(c * _L, _L)
      tl = jnp.full((_L,), c * _L, jnp.int32) + lane
      gi = lax.shift_right_logical(tl, three_i)
      ri = tl & seven_i
      x0 = plsc.load_gather(buf, (gi, ri, zero_i))
      x1 = plsc.load_gather(buf, (gi, ri, one_i))
      is_early = pos_v[ph, sl_ix] < half
      sel_v[ph, sl_ix] = jnp.where(is_early, e_sel, l_sel)
      tgt_v[ph, sl_ix] = (jnp.where(is_early, zero_i, four_i)
                          + jnp.where(x0 > zero_f, two_i, zero_i)
                          + jnp.where(x1 > zero_f, one_i, zero_i))
    if ph + 2 < batch:
      h[ph % 2] = fire(ph + 2)

  pltpu.sync_copy(sel_v, sel_out.at[:, pl.ds(col0, stripe)])
  pltpu.sync_copy(tgt_v, tgt_out.at[:, pl.ds(col0, stripe)])


def kernel(x, positions, seq_len, position_sigs, pos_early, pos_late):
  b, s, d = x.shape
  n = b * s
  t_tiles = position_sigs.shape[0]
  stripe = s // _NW

  x4 = x.reshape(n // 8, 8, d)
  pos2d = positions.astype(jnp.int32)
  half = jnp.full((_L,), (jnp.asarray(seq_len, jnp.int32) + 1) // 2,
                  dtype=jnp.int32)
  tables = jnp.concatenate(
      [position_sigs.reshape(t_tiles * 16), pos_early, pos_late])

  mesh = plsc.VectorSubcoreMesh(core_axis_name="c", subcore_axis_name="s",
                                num_cores=2, num_subcores=16)
  out_i32 = jax.ShapeDtypeStruct((b, s), jnp.int32)
  fn = pl.kernel(
      functools.partial(_router_body, b, s, d, t_tiles),
      out_type=[out_i32, out_i32],
      mesh=mesh,
      compiler_params=pltpu.CompilerParams(needs_layout_passes=False),
      scratch_types=[
          pltpu.VMEM((stripe // 8, 8, 128), jnp.float32),
          pltpu.VMEM((stripe // 8, 8, 128), jnp.float32),
          pltpu.VMEM((b, stripe), jnp.int32),
          pltpu.VMEM((b, stripe), jnp.int32),
          pltpu.VMEM((b, stripe), jnp.int32),
          pltpu.VMEM(((t_tiles + 2) * _L,), jnp.float32),
          pltpu.VMEM((_L,), jnp.int32),
          pltpu.SemaphoreType.DMA,
          pltpu.SemaphoreType.DMA,
          pltpu.SemaphoreType.DMA,
      ],
  )
  return tuple(fn(x4, pos2d, half, tables))
